# Initial kernel scaffold; baseline (speedup 1.0000x reference)
import jax
import jax.numpy as jnp
from jax.experimental import pallas as pl


def kernel(edge_feats, neighbors, W, b):
    raise NotImplementedError("write your pallas kernel here")


# speedup vs baseline: 1.7495x; 1.7495x over previous
import functools

import jax
import jax.numpy as jnp
from jax import lax
from jax.experimental import pallas as pl
from jax.experimental.pallas import tpu as pltpu
from jax.experimental.pallas import tpu_sc as plsc

E = 20000
K = 8
D = 512

NC = 2
NS = 16
NW = NC * NS
KP = K + 1
G = 8
NGT = E // G
LANES = 16


def _sc_gather_sum(x, idx_flat):
    mesh = plsc.VectorSubcoreMesh(core_axis_name="c", subcore_axis_name="s")

    @functools.partial(
        pl.kernel,
        out_type=jax.ShapeDtypeStruct((E, D), jnp.float32),
        mesh=mesh,
        scratch_types=[
            pltpu.VMEM((G * KP,), jnp.int32),
            pltpu.VMEM((G * KP, D), jnp.float32),
            pltpu.VMEM((G, D), jnp.float32),
            pltpu.SemaphoreType.DMA,
        ],
    )
    def sc_fn(x_hbm, idx_hbm, out_hbm, idx_v, rows_v, out_v, sem):
        wid = lax.axis_index("s") * NC + lax.axis_index("c")
        ng = (NGT - wid + NW - 1) // NW

        def group(n, carry):
            g = wid + n * NW
            pltpu.sync_copy(idx_hbm.at[pl.ds(g * (G * KP), G * KP)], idx_v)
            pltpu.async_copy(x_hbm.at[idx_v], rows_v, sem).wait()
            for i in range(G):
                def slice_body(s, c):
                    d = pl.ds(pl.multiple_of(s * LANES, LANES), LANES)
                    acc = rows_v[i * KP, d]
                    for j in range(1, KP):
                        acc = acc + rows_v[i * KP + j, d]
                    out_v[i, d] = acc
                    return c
                lax.fori_loop(0, D // LANES, slice_body, 0)
            pltpu.sync_copy(out_v, out_hbm.at[pl.ds(g * G, G)])
            return carry

        lax.fori_loop(0, ng, group, 0)

    return sc_fn(x, idx_flat)


def _mm_body(s_ref, w_ref, b_ref, o_ref):
    acc = lax.dot_general(
        s_ref[...], w_ref[...], (((1,), (1,)), ((), ())),
        preferred_element_type=jnp.float32,
    )
    o_ref[...] = acc + (K + 1.0) * b_ref[...]


def _tc_matmul(s, w, b):
    BM = 2000
    return pl.pallas_call(
        _mm_body,
        grid=(E // BM,),
        in_specs=[
            pl.BlockSpec((BM, D), lambda i: (i, 0)),
            pl.BlockSpec((D, D), lambda i: (0, 0)),
            pl.BlockSpec((1, D), lambda i: (0, 0)),
        ],
        out_specs=pl.BlockSpec((BM, D), lambda i: (i, 0)),
        out_shape=jax.ShapeDtypeStruct((E, D), jnp.float32),
    )(s, w, b.reshape(1, D))


def kernel(edge_feats, neighbors, W, b):
    self_idx = jnp.arange(E, dtype=jnp.int32)[:, None]
    idx_flat = jnp.concatenate(
        [self_idx, neighbors.astype(jnp.int32)], axis=1
    ).reshape(E * KP)
    s = _sc_gather_sum(edge_feats, idx_flat)
    return _tc_matmul(s, W, b)


# speedup vs baseline: 3.0530x; 1.7451x over previous
import functools

import jax
import jax.numpy as jnp
from jax import lax
from jax.experimental import pallas as pl
from jax.experimental.pallas import tpu as pltpu
from jax.experimental.pallas import tpu_sc as plsc

E = 20000
K = 8
D = 512

NC = 2
NS = 16
NW = NC * NS
KP = K + 1
G = 8
NGT = E // G
LANES = 16


GKP = G * KP
NT = 80


def _sc_gather_sum(x, idx_flat):
    mesh = plsc.VectorSubcoreMesh(core_axis_name="c", subcore_axis_name="s")

    @functools.partial(
        pl.kernel,
        out_type=jax.ShapeDtypeStruct((E, D), jnp.float32),
        mesh=mesh,
        scratch_types=[
            pltpu.VMEM((2, GKP), jnp.int32),
            pltpu.VMEM((2, GKP, D), jnp.float32),
            pltpu.VMEM((2, G, D), jnp.float32),
            pltpu.SemaphoreType.DMA,
            pltpu.SemaphoreType.DMA,
            pltpu.SemaphoreType.DMA,
            pltpu.SemaphoreType.DMA,
            pltpu.SemaphoreType.DMA,
            pltpu.SemaphoreType.DMA,
        ],
    )
    def sc_fn(x_hbm, idx_hbm, out_hbm, idx_v, rows_v, out_v,
              si0, si1, sr0, sr1, so0, so1):
        wid = lax.axis_index("s") * NC + lax.axis_index("c")
        s_idx = (si0, si1)
        s_rows = (sr0, sr1)
        s_out = (so0, so1)

        def gof(n):
            return jnp.minimum(wid + n * NW, NGT - 1)

        def idx_dma(n, p):
            return pltpu.make_async_copy(
                idx_hbm.at[pl.ds(gof(n) * GKP, GKP)], idx_v.at[p], s_idx[p])

        def rows_dma(p):
            return pltpu.make_async_copy(
                x_hbm.at[idx_v.at[p]], rows_v.at[p], s_rows[p])

        def out_dma(n, p):
            return pltpu.make_async_copy(
                out_v.at[p], out_hbm.at[pl.ds(gof(n) * G, G)], s_out[p])

        def compute(n, p):
            def slice_body(s, c):
                d = pl.ds(pl.multiple_of(s * LANES, LANES), LANES)
                for i in range(G):
                    r = [rows_v[p, i * KP + j, d] for j in range(KP)]
                    while len(r) > 1:
                        nxt = [r[k] + r[k + 1] for k in range(0, len(r) - 1, 2)]
                        if len(r) % 2:
                            nxt.append(r[-1])
                        r = nxt
                    out_v[p, i, d] = r[0]
                return c
            lax.fori_loop(0, D // LANES, slice_body, 0)

        idx_dma(0, 0).start()
        idx_dma(1, 1).start()
        idx_dma(0, 0).wait()
        rows_dma(0).start()

        def pair(m, carry):
            for p in (0, 1):
                n = 2 * m + p
                if p == 0:
                    idx_dma(n + 1, 1).wait()
                    rows_dma(1).start()
                else:
                    @pl.when(m < NT // 2 - 1)
                    def _():
                        idx_dma(n + 1, 0).wait()
                        rows_dma(0).start()
                rows_dma(p).wait()
                @pl.when(m < NT // 2 - 1)
                def _():
                    idx_dma(n + 2, p).start()
                @pl.when(m >= 1)
                def _():
                    out_dma(n - 2, p).wait()
                compute(n, p)
                out_dma(n, p).start()
            return carry

        lax.fori_loop(0, NT // 2, pair, 0)
        out_dma(NT - 2, 0).wait()
        out_dma(NT - 1, 1).wait()

    return sc_fn(x, idx_flat)


def _mm_body(s_ref, w_ref, b_ref, o_ref):
    acc = lax.dot_general(
        s_ref[...], w_ref[...], (((1,), (1,)), ((), ())),
        preferred_element_type=jnp.float32,
    )
    o_ref[...] = acc + (K + 1.0) * b_ref[...]


def _tc_matmul(s, w, b):
    BM = 2000
    return pl.pallas_call(
        _mm_body,
        grid=(E // BM,),
        in_specs=[
            pl.BlockSpec((BM, D), lambda i: (i, 0)),
            pl.BlockSpec((D, D), lambda i: (0, 0)),
            pl.BlockSpec((1, D), lambda i: (0, 0)),
        ],
        out_specs=pl.BlockSpec((BM, D), lambda i: (i, 0)),
        out_shape=jax.ShapeDtypeStruct((E, D), jnp.float32),
    )(s, w, b.reshape(1, D))


def kernel(edge_feats, neighbors, W, b):
    self_idx = jnp.arange(E, dtype=jnp.int32)[:, None]
    idx_flat = jnp.concatenate(
        [self_idx, neighbors.astype(jnp.int32)], axis=1
    ).reshape(E * KP)
    s = _sc_gather_sum(edge_feats, idx_flat)
    return _tc_matmul(s, W, b)
